# Initial kernel scaffold; baseline (speedup 1.0000x reference)
import jax
import jax.numpy as jnp
from jax.experimental import pallas as pl


def kernel(sequence, vector, sequence_mask):
    raise NotImplementedError("write your pallas kernel here")


# speedup vs baseline: 1.9295x; 1.9295x over previous
import jax
import jax.numpy as jnp
from jax.experimental import pallas as pl
from jax.experimental.pallas import tpu as pltpu

_NEG_INF = -1e30


def _attn_kernel(vec_ref, mask_ref, seq_ref, pooled_ref, w_ref):
    seq = seq_ref[0]
    vec = vec_ref[0]
    scores = jax.lax.dot_general(
        vec, seq, (((1,), (1,)), ((), ())),
        preferred_element_type=jnp.float32)
    masked = jnp.where(mask_ref[0] > 0, scores, _NEG_INF)
    m = jnp.max(masked, axis=-1, keepdims=True)
    e = jnp.exp(masked - m)
    s = jnp.sum(e, axis=-1, keepdims=True)
    w = e / s
    w_ref[0] = w
    pooled_ref[0] = jax.lax.dot_general(
        w, seq, (((1,), (0,)), ((), ())),
        preferred_element_type=jnp.float32)


def kernel(sequence, vector, sequence_mask):
    B, L, D = sequence.shape
    vec3 = vector.reshape(B, 1, D)
    mask3 = sequence_mask.reshape(B, 1, L)
    pooled, weights = pl.pallas_call(
        _attn_kernel,
        grid=(B,),
        in_specs=[
            pl.BlockSpec((1, 1, D), lambda b: (b, 0, 0)),
            pl.BlockSpec((1, 1, L), lambda b: (b, 0, 0)),
            pl.BlockSpec((1, L, D), lambda b: (b, 0, 0)),
        ],
        out_specs=[
            pl.BlockSpec((1, 1, D), lambda b: (b, 0, 0)),
            pl.BlockSpec((1, 1, L), lambda b: (b, 0, 0)),
        ],
        out_shape=[
            jax.ShapeDtypeStruct((B, 1, D), jnp.float32),
            jax.ShapeDtypeStruct((B, 1, L), jnp.float32),
        ],
        compiler_params=pltpu.CompilerParams(
            dimension_semantics=("parallel",),
            vmem_limit_bytes=48 * 1024 * 1024,
        ),
        name="uni_dir_attention",
    )(vec3, mask3, sequence)
    return pooled.reshape(B, D), weights.reshape(B, L)
